# baseline (speedup 1.0000x reference)
import functools

import jax
import jax.numpy as jnp
from jax import lax
from jax.experimental import pallas as pl
from jax.experimental.pallas import tpu as pltpu
from jax.experimental.pallas import tpu_sc as plsc

D_MODEL = 1024
NUM_EMB = 4

_NC = 2
_NS = 16
_NW = _NC * _NS

_D2 = 2 * D_MODEL
_NPAIR = 4 * 8192 // 2
_BPW = _NPAIR // _NW
_R = 16
_NCH = _BPW // _R
_NBUF = 3
_NEMB2 = NUM_EMB * NUM_EMB


def _sc_body(pids_hbm, table_hbm, out_hbm, pidx_v, bufs, *sems):
    sid = lax.axis_index("s")
    wid = sid * _NC + lax.axis_index("c")
    base = wid * _BPW

    pltpu.sync_copy(pids_hbm.at[pl.ds(base, _BPW)], pidx_v)
    off = wid * _NEMB2
    for j in range(_BPW // 16):
        sl = pl.ds(j * 16, 16)
        pidx_v[sl] = pidx_v[sl] + off

    gsems = sems[:_NBUF]
    ssems = sems[_NBUF:]
    gd = [None] * _NCH
    sd = [None] * _NCH

    def start_gather(i):
        b = i % _NBUF
        gd[i] = pltpu.async_copy(
            table_hbm.at[pidx_v.at[pl.ds(i * _R, _R)]], bufs.at[b], gsems[b])

    def start_scatter(i):
        b = i % _NBUF
        sd[i] = pltpu.async_copy(
            bufs.at[b], out_hbm.at[pl.ds(base + i * _R, _R)], ssems[b])

    for g in range(min(_NBUF - 1, _NCH)):
        start_gather(g)
    for i in range(_NCH):
        g = i + _NBUF - 1
        if g < _NCH:
            if g - _NBUF >= 0:
                sd[g - _NBUF].wait()
            start_gather(g)
        gd[i].wait()
        start_scatter(i)
    for i in range(max(0, _NCH - _NBUF), _NCH):
        sd[i].wait()


@jax.jit
def _sc_gather(pids, pair_table):
    mesh = plsc.VectorSubcoreMesh(
        core_axis_name="c", subcore_axis_name="s",
        num_cores=_NC, num_subcores=_NS)
    f = functools.partial(
        pl.kernel,
        out_type=jax.ShapeDtypeStruct((_NPAIR, _D2), jnp.float32),
        mesh=mesh,
        scratch_types=[
            pltpu.VMEM((_BPW,), jnp.int32),
            pltpu.VMEM((_NBUF, _R, _D2), jnp.float32),
        ] + [pltpu.SemaphoreType.DMA] * (2 * _NBUF),
    )(_sc_body)
    return f(pids, pair_table)


def kernel(postion_ids, table):
    B, S = postion_ids.shape
    ids_flat = postion_ids.reshape(B * S).astype(jnp.int32)
    pids = ids_flat[0::2] * NUM_EMB + ids_flat[1::2]
    pt = jnp.concatenate(
        [jnp.repeat(table, NUM_EMB, axis=0), jnp.tile(table, (NUM_EMB, 1))],
        axis=1)
    pt_rep = jnp.tile(pt, (_NW, 1))
    out = _sc_gather(pids, pt_rep)
    return out.reshape(B, S, D_MODEL)
